# Initial kernel scaffold; baseline (speedup 1.0000x reference)
import jax
import jax.numpy as jnp
from jax.experimental import pallas as pl


def kernel(x, fw_adj, bw_adj, fw_W0, fw_W1, bw_W0, bw_W1):
    raise NotImplementedError("write your pallas kernel here")


# speedup vs baseline: 2.0015x; 2.0015x over previous
import functools
import math

import jax
import jax.numpy as jnp
from jax import lax
from jax.experimental import pallas as pl
from jax.experimental.pallas import tpu as pltpu
from jax.experimental.pallas import tpu_sc as plsc

_N = 10000
_S = 32
_H = 128


@functools.cache
def _make_sc_gather(M: int, D: int, C: int = 80):
    info = plsc.get_sparse_core_info()
    nc, ns = info.num_cores, info.num_subcores
    nw = nc * ns
    per_w = M // nw
    assert M % nw == 0 and per_w % C == 0 and per_w % 8 == 0 and C % 8 == 0
    nchunks = per_w // C
    mesh = plsc.VectorSubcoreMesh(core_axis_name="c", subcore_axis_name="s")

    @functools.partial(
        pl.kernel,
        mesh=mesh,
        out_type=jax.ShapeDtypeStruct((M, D), jnp.float32),
        scratch_types=[
            pltpu.VMEM((C,), jnp.int32),
            pltpu.VMEM((C, D), jnp.float32),
            pltpu.SemaphoreType.DMA,
        ],
    )
    def gather_k(table_hbm, idx_hbm, out_hbm, idx_v, rows_v, sem):
        wid = lax.axis_index("s") * nc + lax.axis_index("c")
        base = wid * per_w

        def step(j, carry):
            b = pl.multiple_of(base + j * C, 8)
            pltpu.sync_copy(idx_hbm.at[pl.ds(b, C)], idx_v)
            pltpu.async_copy(table_hbm.at[idx_v], rows_v, sem).wait()
            pltpu.sync_copy(rows_v, out_hbm.at[pl.ds(b, C)])
            return carry

        lax.fori_loop(0, nchunks, step, 0)

    return gather_k


def _attn_pallas(self_h, g_sm, w, d: int, hh: int, bn: int = 200):
    n = self_h.shape[0]
    inv = 1.0 / math.sqrt(float(d))

    def body(self_ref, g_ref, w_ref, out_ref):
        wm = w_ref[...]
        sh = self_ref[...]
        q = jnp.dot(sh, wm[:, :d], preferred_element_type=jnp.float32)
        sp = jnp.dot(sh, wm[:, d:d + hh], preferred_element_type=jnp.float32)
        cols = []
        for s in range(_S):
            cols.append(jnp.sum(q * g_ref[s], axis=1, keepdims=True))
        sc = jnp.concatenate(cols, axis=1) * inv
        m = jnp.max(sc, axis=1, keepdims=True)
        e = jnp.exp(sc - m)
        a = e / jnp.sum(e, axis=1, keepdims=True)
        agg = a[:, 0:1] * g_ref[0]
        for s in range(1, _S):
            agg = agg + a[:, s:s + 1] * g_ref[s]
        np_ = jnp.dot(agg, wm[:, d + hh:], preferred_element_type=jnp.float32)
        out_ref[...] = jnp.maximum(jnp.concatenate([sp, np_], axis=1), 0.0)

    return pl.pallas_call(
        body,
        grid=(n // bn,),
        in_specs=[
            pl.BlockSpec((bn, d), lambda i: (i, 0)),
            pl.BlockSpec((_S, bn, d), lambda i: (0, i, 0)),
            pl.BlockSpec((d, d + 2 * hh), lambda i: (0, 0)),
        ],
        out_specs=pl.BlockSpec((bn, 2 * hh), lambda i: (i, 0)),
        out_shape=jax.ShapeDtypeStruct((n, 2 * hh), jnp.float32),
    )(self_h, g_sm, w)


def kernel(x, fw_adj, bw_adj, fw_W0, fw_W1, bw_W0, bw_W1):
    fw_nb = fw_adj[:_N, :_S].astype(jnp.int32)
    bw_nb = bw_adj[:_N, :_S].astype(jnp.int32)
    fw_idx = fw_nb.T.reshape(-1)
    bw_idx = bw_nb.T.reshape(-1)

    m1 = _N * _S
    idx0 = jnp.concatenate([fw_idx, bw_idx])
    g0 = _make_sc_gather(2 * m1, _H)(x, idx0)
    g0_fw = g0[:m1].reshape(_S, _N, _H)
    g0_bw = g0[m1:].reshape(_S, _N, _H)

    h0_fw = _attn_pallas(x, g0_fw, fw_W0, _H, _H)
    h0_bw = _attn_pallas(x, g0_bw, bw_W0, _H, _H)

    gather1 = _make_sc_gather(m1, 2 * _H)
    g1_fw = gather1(h0_fw, fw_idx).reshape(_S, _N, 2 * _H)
    g1_bw = gather1(h0_bw, bw_idx).reshape(_S, _N, 2 * _H)

    h1_fw = _attn_pallas(h0_fw, g1_fw, fw_W1, 2 * _H, _H)
    h1_bw = _attn_pallas(h0_bw, g1_bw, bw_W1, 2 * _H, _H)

    return jnp.concatenate([h1_fw, h1_bw], axis=-1)
